# baseline (speedup 1.0000x reference)
import functools

import jax
import jax.numpy as jnp
from jax import lax
from jax.experimental import pallas as pl
from jax.experimental.pallas import tpu as pltpu
from jax.experimental.pallas import tpu_sc as plsc

N_NODES = 10000
N_EDGES = 320000
D_FEAT = 128
D_EDGE = 16
HIDDEN = 64

NUM_CORES = 2
NUM_SUBCORES = 16
NUM_TILES = NUM_CORES * NUM_SUBCORES

CHUNK = 128
E_PAD = 327680
PAD = E_PAD - N_EDGES
EDGES_PER_TILE = E_PAD // NUM_TILES
CHUNKS_PER_TILE = EDGES_PER_TILE // CHUNK
NUM_CHUNK_ROWS = E_PAD // CHUNK
N_ACC = 10016
ROWS_PER_SUBCORE = N_ACC // NUM_SUBCORES

_SC_MESH = dict(core_axis_name="c", subcore_axis_name="s",
                num_cores=NUM_CORES, num_subcores=NUM_SUBCORES)
_SC_PARAMS = pltpu.CompilerParams(use_tc_tiling_on_sc=False)


def _xa_body(x_ref, w_ref, o_ref):
    o_ref[...] = jnp.dot(x_ref[...], w_ref[...],
                         preferred_element_type=jnp.float32)


def _compute_xa(x, w1a):
    return pl.pallas_call(
        _xa_body,
        out_shape=jax.ShapeDtypeStruct((N_NODES, HIDDEN), jnp.float32),
    )(x, w1a)


G_SUP_CH = 4
G_SUP_E = G_SUP_CH * CHUNK
G_SUPERS = CHUNKS_PER_TILE // G_SUP_CH


def _gather_body(xa_hbm, src2_hbm, g_hbm, idx2, big0, big1,
                 gs0, gs1, ss0, ss1):
    wid = lax.axis_index("s") * NUM_CORES + lax.axis_index("c")
    crow = wid * CHUNKS_PER_TILE
    base = wid * EDGES_PER_TILE
    bigs = (big0, big1)
    gsems = (gs0, gs1)
    ssems = (ss0, ss1)

    pltpu.sync_copy(src2_hbm.at[pl.ds(crow, CHUNKS_PER_TILE)], idx2)

    @pl.loop(0, G_SUPERS // 2)
    def _(p):
        for b in range(2):
            s = p * 2 + b

            @pl.when(p > 0)
            def _():
                pltpu.make_async_copy(
                    g_hbm.at[pl.ds(base, G_SUP_E)], bigs[b], ssems[b]).wait()

            descs = []
            for j in range(G_SUP_CH):
                c = s * G_SUP_CH + j
                descs.append(pltpu.async_copy(
                    xa_hbm.at[idx2.at[c]],
                    bigs[b].at[pl.ds(j * CHUNK, CHUNK)], gsems[b]))
            for d in descs:
                d.wait()
            pltpu.async_copy(bigs[b],
                             g_hbm.at[pl.ds(base + s * G_SUP_E, G_SUP_E)],
                             ssems[b])

    for b in range(2):
        pltpu.make_async_copy(
            g_hbm.at[pl.ds(base, G_SUP_E)], bigs[b], ssems[b]).wait()


def _sc_gather(xa, src2):
    k = pl.kernel(
        _gather_body,
        out_type=jax.ShapeDtypeStruct((E_PAD, HIDDEN), jnp.float32),
        mesh=plsc.VectorSubcoreMesh(**_SC_MESH),
        scratch_types=[
            pltpu.VMEM((CHUNKS_PER_TILE, CHUNK), jnp.int32),
            pltpu.VMEM((G_SUP_E, HIDDEN), jnp.float32),
            pltpu.VMEM((G_SUP_E, HIDDEN), jnp.float32),
            pltpu.SemaphoreType.DMA,
            pltpu.SemaphoreType.DMA,
            pltpu.SemaphoreType.DMA,
            pltpu.SemaphoreType.DMA,
        ],
        compiler_params=_SC_PARAMS,
    )
    return k(xa, src2)


N_E2 = E_PAD // 2
BLOCK_R = 2048
_INV_H = 1.0 / HIDDEN


def _ln2(h, mstat, gamma, beta):
    mu = jnp.dot(h, mstat, preferred_element_type=jnp.float32)
    d = h - mu
    var = jnp.dot(d * d, mstat, preferred_element_type=jnp.float32)
    return d * lax.rsqrt(var + 1e-5) * gamma + beta


def _mlp_body(g_ref, ea_ref, w1b_ref, mstat_ref, b1_ref, g1_ref, beta1_ref,
              w2_ref, b2_ref, g2_ref, beta2_ref, o_ref):
    mstat = mstat_ref[...]
    ea = jnp.dot(ea_ref[...], w1b_ref[...],
                 preferred_element_type=jnp.float32)
    h = jnp.maximum(g_ref[...] + ea + b1_ref[...], 0.0)
    h = _ln2(h, mstat, g1_ref[...], beta1_ref[...])
    h = jnp.dot(h, w2_ref[...],
                preferred_element_type=jnp.float32) + b2_ref[...]
    h = jnp.maximum(h, 0.0)
    o_ref[...] = _ln2(h, mstat, g2_ref[...], beta2_ref[...])


def _edge_mlp(g2, ea2, w1b_blk, mstat, b1_blk, g1_blk, beta1_blk,
              w2_blk, b2_blk, g2_blk, beta2_blk):
    vec = pl.BlockSpec((1, 2 * HIDDEN), lambda i: (0, 0))
    return pl.pallas_call(
        _mlp_body,
        grid=(N_E2 // BLOCK_R,),
        in_specs=[
            pl.BlockSpec((BLOCK_R, 2 * HIDDEN), lambda i: (i, 0)),
            pl.BlockSpec((BLOCK_R, 2 * D_EDGE), lambda i: (i, 0)),
            pl.BlockSpec((2 * D_EDGE, 2 * HIDDEN), lambda i: (0, 0)),
            pl.BlockSpec((2 * HIDDEN, 2 * HIDDEN), lambda i: (0, 0)),
            vec, vec, vec,
            pl.BlockSpec((2 * HIDDEN, 2 * HIDDEN), lambda i: (0, 0)),
            vec, vec, vec,
        ],
        out_specs=pl.BlockSpec((BLOCK_R, 2 * HIDDEN), lambda i: (i, 0)),
        out_shape=jax.ShapeDtypeStruct((N_E2, 2 * HIDDEN), jnp.float32),
    )(g2, ea2, w1b_blk, mstat, b1_blk, g1_blk, beta1_blk,
      w2_blk, b2_blk, g2_blk, beta2_blk)


SUP_CH = 4
SUP_E = SUP_CH * CHUNK
SUPERS_PER_TILE = CHUNKS_PER_TILE // SUP_CH


def _scatter_body(h2_hbm, dst2_hbm, zeros_hbm, out_hbm,
                  idx2, big0, big1, ls0, ls1, accum):
    core = lax.axis_index("c")
    sid = lax.axis_index("s")
    wid = sid * NUM_CORES + core
    crow = wid * CHUNKS_PER_TILE
    base = wid * EDGES_PER_TILE
    stripe = sid * ROWS_PER_SUBCORE
    bigs = (big0, big1)
    lsems = (ls0, ls1)

    pltpu.sync_copy(zeros_hbm, accum.at[pl.ds(stripe, ROWS_PER_SUBCORE)])
    pltpu.sync_copy(dst2_hbm.at[pl.ds(crow, CHUNKS_PER_TILE)], idx2)
    plsc.subcore_barrier()

    @pl.loop(0, SUPERS_PER_TILE // 2)
    def _(p):
        loads = []
        for b in range(2):
            s = p * 2 + b
            loads.append(pltpu.async_copy(
                h2_hbm.at[pl.ds(base + s * SUP_E, SUP_E)], bigs[b], lsems[b]))
        for b in range(2):
            s = p * 2 + b
            loads[b].wait()
            for j in range(SUP_CH):
                pltpu.sync_copy(bigs[b].at[pl.ds(j * CHUNK, CHUNK)],
                                accum.at[idx2.at[s * SUP_CH + j]], add=True)

    plsc.subcore_barrier()
    pltpu.sync_copy(accum.at[pl.ds(stripe, ROWS_PER_SUBCORE)],
                    out_hbm.at[pl.ds(core * N_ACC + stripe,
                                     ROWS_PER_SUBCORE)])


def _sc_scatter_add(h2, dst2, zeros):
    k = pl.kernel(
        _scatter_body,
        out_type=jax.ShapeDtypeStruct((NUM_CORES * N_ACC, HIDDEN),
                                      jnp.float32),
        mesh=plsc.VectorSubcoreMesh(**_SC_MESH),
        scratch_types=[
            pltpu.VMEM((CHUNKS_PER_TILE, CHUNK), jnp.int32),
            pltpu.VMEM((SUP_E, HIDDEN), jnp.float32),
            pltpu.VMEM((SUP_E, HIDDEN), jnp.float32),
            pltpu.SemaphoreType.DMA,
            pltpu.SemaphoreType.DMA,
            pltpu.VMEM_SHARED((N_ACC, HIDDEN), jnp.float32),
        ],
        compiler_params=_SC_PARAMS,
    )
    return k(h2, dst2, zeros)


def _sum_body(p_ref, o_ref):
    o_ref[...] = p_ref[0, :N_NODES, :] + p_ref[1, :N_NODES, :]


def _sum_partials(partials):
    return pl.pallas_call(
        _sum_body,
        out_shape=jax.ShapeDtypeStruct((N_NODES, HIDDEN), jnp.float32),
    )(partials.reshape(NUM_CORES, N_ACC, HIDDEN))


def _blockdiag2(w):
    k, n = w.shape
    z = jnp.zeros((k, n), w.dtype)
    return jnp.concatenate([jnp.concatenate([w, z], axis=1),
                            jnp.concatenate([z, w], axis=1)], axis=0)


def _dup(v):
    return jnp.concatenate([v, v]).reshape(1, 2 * HIDDEN)


def kernel(x, edge_index, edge_attr, W1, b1, g1, beta1, W2, b2, g2, beta2):
    src = jnp.concatenate([edge_index[0],
                           jnp.zeros((PAD,), jnp.int32)])
    dst = jnp.concatenate([edge_index[1],
                           jnp.full((PAD,), N_NODES, jnp.int32)])
    ea_p = jnp.concatenate([edge_attr,
                            jnp.zeros((PAD, D_EDGE), jnp.float32)])
    w1a = W1[:D_FEAT]
    w1b = W1[D_FEAT:]
    zeros = jnp.zeros((ROWS_PER_SUBCORE, HIDDEN), jnp.float32)

    xa = _compute_xa(x, w1a)
    g = _sc_gather(xa, src.reshape(NUM_CHUNK_ROWS, CHUNK))
    mstat = _blockdiag2(jnp.full((HIDDEN, HIDDEN), _INV_H, jnp.float32))
    h2 = _edge_mlp(g.reshape(N_E2, 2 * HIDDEN),
                   ea_p.reshape(N_E2, 2 * D_EDGE),
                   _blockdiag2(w1b), mstat, _dup(b1), _dup(g1), _dup(beta1),
                   _blockdiag2(W2), _dup(b2), _dup(g2), _dup(beta2))
    partials = _sc_scatter_add(h2.reshape(E_PAD, HIDDEN),
                               dst.reshape(NUM_CHUNK_ROWS, CHUNK), zeros)
    return _sum_partials(partials)


# speedup vs baseline: 1.2359x; 1.2359x over previous
import functools

import jax
import jax.numpy as jnp
from jax import lax
from jax.experimental import pallas as pl
from jax.experimental.pallas import tpu as pltpu
from jax.experimental.pallas import tpu_sc as plsc

N_NODES = 10000
N_EDGES = 320000
D_FEAT = 128
D_EDGE = 16
HIDDEN = 64

NUM_CORES = 2
NUM_SUBCORES = 16
NUM_TILES = NUM_CORES * NUM_SUBCORES

CHUNK = 128
E_PAD = 327680
PAD = E_PAD - N_EDGES
EDGES_PER_TILE = E_PAD // NUM_TILES
CHUNKS_PER_TILE = EDGES_PER_TILE // CHUNK
NUM_CHUNK_ROWS = E_PAD // CHUNK
N_ACC = 10016
ROWS_PER_SUBCORE = N_ACC // NUM_SUBCORES

_SC_MESH = dict(core_axis_name="c", subcore_axis_name="s",
                num_cores=NUM_CORES, num_subcores=NUM_SUBCORES)
_SC_PARAMS = pltpu.CompilerParams(use_tc_tiling_on_sc=False)


def _xa_body(x_ref, w_ref, o_ref):
    o_ref[...] = jnp.dot(x_ref[...], w_ref[...],
                         preferred_element_type=jnp.float32)


def _compute_xa(x, w1a):
    return pl.pallas_call(
        _xa_body,
        out_shape=jax.ShapeDtypeStruct((N_NODES, HIDDEN), jnp.float32),
    )(x, w1a)


G_SUP_CH = 4
G_SUP_E = G_SUP_CH * CHUNK
G_SUPERS = CHUNKS_PER_TILE // G_SUP_CH


def _gather_body(xa_hbm, src2_hbm, g_hbm, idx2, big0, big1,
                 gs0, gs1, ss0, ss1):
    wid = lax.axis_index("s") * NUM_CORES + lax.axis_index("c")
    crow = wid * CHUNKS_PER_TILE
    base = wid * EDGES_PER_TILE
    bigs = (big0, big1)
    gsems = (gs0, gs1)
    ssems = (ss0, ss1)

    pltpu.sync_copy(src2_hbm.at[pl.ds(crow, CHUNKS_PER_TILE)], idx2)

    @pl.loop(0, G_SUPERS // 2)
    def _(p):
        for b in range(2):
            s = p * 2 + b

            @pl.when(p > 0)
            def _():
                pltpu.make_async_copy(
                    g_hbm.at[pl.ds(base, G_SUP_E)], bigs[b], ssems[b]).wait()

            descs = []
            for j in range(G_SUP_CH):
                c = s * G_SUP_CH + j
                descs.append(pltpu.async_copy(
                    xa_hbm.at[idx2.at[c]],
                    bigs[b].at[pl.ds(j * CHUNK, CHUNK)], gsems[b]))
            for d in descs:
                d.wait()
            pltpu.async_copy(bigs[b],
                             g_hbm.at[pl.ds(base + s * G_SUP_E, G_SUP_E)],
                             ssems[b])

    for b in range(2):
        pltpu.make_async_copy(
            g_hbm.at[pl.ds(base, G_SUP_E)], bigs[b], ssems[b]).wait()


def _sc_gather(xa, src2):
    k = pl.kernel(
        _gather_body,
        out_type=jax.ShapeDtypeStruct((E_PAD, HIDDEN), jnp.float32),
        mesh=plsc.VectorSubcoreMesh(**_SC_MESH),
        scratch_types=[
            pltpu.VMEM((CHUNKS_PER_TILE, CHUNK), jnp.int32),
            pltpu.VMEM((G_SUP_E, HIDDEN), jnp.float32),
            pltpu.VMEM((G_SUP_E, HIDDEN), jnp.float32),
            pltpu.SemaphoreType.DMA,
            pltpu.SemaphoreType.DMA,
            pltpu.SemaphoreType.DMA,
            pltpu.SemaphoreType.DMA,
        ],
        compiler_params=_SC_PARAMS,
    )
    return k(xa, src2)


N_E2 = E_PAD // 2
BLOCK_R = 2048
_INV_H = 1.0 / HIDDEN


def _ln2(h, mstat, gamma, beta):
    mu = jnp.dot(h, mstat, preferred_element_type=jnp.float32)
    d = h - mu
    var = jnp.dot(d * d, mstat, preferred_element_type=jnp.float32)
    return d * lax.rsqrt(var + 1e-5) * gamma + beta


def _mlp_body(g_ref, ea_ref, w1b_ref, mstat_ref, b1_ref, g1_ref, beta1_ref,
              w2_ref, b2_ref, g2_ref, beta2_ref, o_ref):
    mstat = mstat_ref[...]
    ea = jnp.dot(ea_ref[...], w1b_ref[...],
                 preferred_element_type=jnp.float32)
    h = jnp.maximum(g_ref[...] + ea + b1_ref[...], 0.0)
    h = _ln2(h, mstat, g1_ref[...], beta1_ref[...])
    h = jnp.dot(h, w2_ref[...],
                preferred_element_type=jnp.float32) + b2_ref[...]
    h = jnp.maximum(h, 0.0)
    o_ref[...] = _ln2(h, mstat, g2_ref[...], beta2_ref[...])


def _edge_mlp(g2, ea2, w1b_blk, mstat, b1_blk, g1_blk, beta1_blk,
              w2_blk, b2_blk, g2_blk, beta2_blk):
    vec = pl.BlockSpec((1, 2 * HIDDEN), lambda i: (0, 0))
    return pl.pallas_call(
        _mlp_body,
        grid=(N_E2 // BLOCK_R,),
        in_specs=[
            pl.BlockSpec((BLOCK_R, 2 * HIDDEN), lambda i: (i, 0)),
            pl.BlockSpec((BLOCK_R, 2 * D_EDGE), lambda i: (i, 0)),
            pl.BlockSpec((2 * D_EDGE, 2 * HIDDEN), lambda i: (0, 0)),
            pl.BlockSpec((2 * HIDDEN, 2 * HIDDEN), lambda i: (0, 0)),
            vec, vec, vec,
            pl.BlockSpec((2 * HIDDEN, 2 * HIDDEN), lambda i: (0, 0)),
            vec, vec, vec,
        ],
        out_specs=pl.BlockSpec((BLOCK_R, 2 * HIDDEN), lambda i: (i, 0)),
        out_shape=jax.ShapeDtypeStruct((N_E2, 2 * HIDDEN), jnp.float32),
    )(g2, ea2, w1b_blk, mstat, b1_blk, g1_blk, beta1_blk,
      w2_blk, b2_blk, g2_blk, beta2_blk)


SUP_CH = 4
SUP_E = SUP_CH * CHUNK
SUPERS_PER_TILE = CHUNKS_PER_TILE // SUP_CH


def _scatter_body(h2_hbm, dst2_hbm, zeros_hbm, out_hbm,
                  idx2, big0, big1, ls0, ls1, accum):
    core = lax.axis_index("c")
    sid = lax.axis_index("s")
    wid = sid * NUM_CORES + core
    crow = wid * CHUNKS_PER_TILE
    base = wid * EDGES_PER_TILE
    stripe = sid * ROWS_PER_SUBCORE
    bigs = (big0, big1)
    lsems = (ls0, ls1)

    pltpu.sync_copy(zeros_hbm, accum.at[pl.ds(stripe, ROWS_PER_SUBCORE)])
    pltpu.sync_copy(dst2_hbm.at[pl.ds(crow, CHUNKS_PER_TILE)], idx2)
    plsc.subcore_barrier()

    @pl.loop(0, SUPERS_PER_TILE // 2)
    def _(p):
        loads = []
        for b in range(2):
            s = p * 2 + b
            loads.append(pltpu.async_copy(
                h2_hbm.at[pl.ds(base + s * SUP_E, SUP_E)], bigs[b], lsems[b]))
        for b in range(2):
            s = p * 2 + b
            loads[b].wait()
            for j in range(SUP_CH):
                pltpu.sync_copy(bigs[b].at[pl.ds(j * CHUNK, CHUNK)],
                                accum.at[idx2.at[s * SUP_CH + j]], add=True)

    plsc.subcore_barrier()
    pltpu.sync_copy(accum.at[pl.ds(stripe, ROWS_PER_SUBCORE)],
                    out_hbm.at[pl.ds(core * N_ACC + stripe,
                                     ROWS_PER_SUBCORE)])


def _sc_scatter_add(h2, dst2, zeros):
    k = pl.kernel(
        _scatter_body,
        out_type=jax.ShapeDtypeStruct((NUM_CORES * N_ACC, HIDDEN),
                                      jnp.float32),
        mesh=plsc.VectorSubcoreMesh(**_SC_MESH),
        scratch_types=[
            pltpu.VMEM((CHUNKS_PER_TILE, CHUNK), jnp.int32),
            pltpu.VMEM((SUP_E, HIDDEN), jnp.float32),
            pltpu.VMEM((SUP_E, HIDDEN), jnp.float32),
            pltpu.SemaphoreType.DMA,
            pltpu.SemaphoreType.DMA,
            pltpu.VMEM_SHARED((N_ACC, HIDDEN), jnp.float32),
        ],
        compiler_params=_SC_PARAMS,
    )
    return k(h2, dst2, zeros)


def _sum_body(p_ref, o_ref):
    o_ref[...] = p_ref[0, :N_NODES, :] + p_ref[1, :N_NODES, :]


def _sum_partials(partials):
    return pl.pallas_call(
        _sum_body,
        out_shape=jax.ShapeDtypeStruct((N_NODES, HIDDEN), jnp.float32),
    )(partials.reshape(NUM_CORES, N_ACC, HIDDEN))


def _blockdiag2(w):
    k, n = w.shape
    z = jnp.zeros((k, n), w.dtype)
    return jnp.concatenate([jnp.concatenate([w, z], axis=1),
                            jnp.concatenate([z, w], axis=1)], axis=0)


def _dup(v):
    return jnp.concatenate([v, v]).reshape(1, 2 * HIDDEN)


def kernel(x, edge_index, edge_attr, W1, b1, g1, beta1, W2, b2, g2, beta2):
    pad_iota = lax.iota(jnp.int32, PAD)
    src = jnp.concatenate([edge_index[0], pad_iota % N_NODES])
    dst = jnp.concatenate([edge_index[1],
                           N_NODES + pad_iota % (N_ACC - N_NODES)])
    ea_p = jnp.concatenate([edge_attr,
                            jnp.zeros((PAD, D_EDGE), jnp.float32)])
    w1a = W1[:D_FEAT]
    w1b = W1[D_FEAT:]
    zeros = jnp.zeros((ROWS_PER_SUBCORE, HIDDEN), jnp.float32)

    xa = _compute_xa(x, w1a)
    g = _sc_gather(xa, src.reshape(NUM_CHUNK_ROWS, CHUNK))
    mstat = _blockdiag2(jnp.full((HIDDEN, HIDDEN), _INV_H, jnp.float32))
    h2 = _edge_mlp(g.reshape(N_E2, 2 * HIDDEN),
                   ea_p.reshape(N_E2, 2 * D_EDGE),
                   _blockdiag2(w1b), mstat, _dup(b1), _dup(g1), _dup(beta1),
                   _blockdiag2(W2), _dup(b2), _dup(g2), _dup(beta2))
    partials = _sc_scatter_add(h2.reshape(E_PAD, HIDDEN),
                               dst.reshape(NUM_CHUNK_ROWS, CHUNK), zeros)
    return _sum_partials(partials)


# speedup vs baseline: 1.5711x; 1.2712x over previous
import functools

import jax
import jax.numpy as jnp
from jax import lax
from jax.experimental import pallas as pl
from jax.experimental.pallas import tpu as pltpu
from jax.experimental.pallas import tpu_sc as plsc

N_NODES = 10000
N_EDGES = 320000
D_FEAT = 128
D_EDGE = 16
HIDDEN = 64

NUM_CORES = 2
NUM_SUBCORES = 16
NUM_TILES = NUM_CORES * NUM_SUBCORES

CHUNK = 125
EDGES_PER_TILE = N_EDGES // NUM_TILES
CHUNKS_PER_TILE = EDGES_PER_TILE // CHUNK
NUM_CHUNK_ROWS = N_EDGES // CHUNK
N_ACC = N_NODES
ROWS_PER_SUBCORE = N_ACC // NUM_SUBCORES

_SC_MESH = dict(core_axis_name="c", subcore_axis_name="s",
                num_cores=NUM_CORES, num_subcores=NUM_SUBCORES)
_SC_PARAMS = pltpu.CompilerParams(use_tc_tiling_on_sc=False)


def _xa_body(x_ref, w_ref, o_ref):
    o_ref[...] = jnp.dot(x_ref[...], w_ref[...],
                         preferred_element_type=jnp.float32)


def _compute_xa(x, w1a):
    return pl.pallas_call(
        _xa_body,
        out_shape=jax.ShapeDtypeStruct((N_NODES, HIDDEN), jnp.float32),
    )(x, w1a)


G_SUP_CH = 4
G_SUP_E = G_SUP_CH * CHUNK
G_SUPERS = CHUNKS_PER_TILE // G_SUP_CH


def _gather_body(xa_hbm, src2_hbm, g_hbm, idx2, big0, big1,
                 gs0, gs1, ss0, ss1):
    wid = lax.axis_index("s") * NUM_CORES + lax.axis_index("c")
    crow = wid * CHUNKS_PER_TILE
    base = wid * EDGES_PER_TILE
    bigs = (big0, big1)
    gsems = (gs0, gs1)
    ssems = (ss0, ss1)

    pltpu.sync_copy(src2_hbm.at[pl.ds(crow, CHUNKS_PER_TILE)], idx2)

    @pl.loop(0, G_SUPERS // 2)
    def _(p):
        for b in range(2):
            s = p * 2 + b

            @pl.when(p > 0)
            def _():
                pltpu.make_async_copy(
                    g_hbm.at[pl.ds(base, G_SUP_E)], bigs[b], ssems[b]).wait()

            descs = []
            for j in range(G_SUP_CH):
                c = s * G_SUP_CH + j
                descs.append(pltpu.async_copy(
                    xa_hbm.at[idx2.at[c]],
                    bigs[b].at[pl.ds(j * CHUNK, CHUNK)], gsems[b]))
            for d in descs:
                d.wait()
            pltpu.async_copy(bigs[b],
                             g_hbm.at[pl.ds(base + s * G_SUP_E, G_SUP_E)],
                             ssems[b])

    for b in range(2):
        pltpu.make_async_copy(
            g_hbm.at[pl.ds(base, G_SUP_E)], bigs[b], ssems[b]).wait()


def _sc_gather(xa, src2):
    k = pl.kernel(
        _gather_body,
        out_type=jax.ShapeDtypeStruct((N_EDGES, HIDDEN), jnp.float32),
        mesh=plsc.VectorSubcoreMesh(**_SC_MESH),
        scratch_types=[
            pltpu.VMEM((CHUNKS_PER_TILE, CHUNK), jnp.int32),
            pltpu.VMEM((G_SUP_E, HIDDEN), jnp.float32),
            pltpu.VMEM((G_SUP_E, HIDDEN), jnp.float32),
            pltpu.SemaphoreType.DMA,
            pltpu.SemaphoreType.DMA,
            pltpu.SemaphoreType.DMA,
            pltpu.SemaphoreType.DMA,
        ],
        compiler_params=_SC_PARAMS,
    )
    return k(xa, src2)


N_E2 = N_EDGES // 2
BLOCK_R = 2000
_INV_H = 1.0 / HIDDEN


def _ln2(h, mstat, gamma, beta):
    mu = jnp.dot(h, mstat, preferred_element_type=jnp.float32)
    d = h - mu
    var = jnp.dot(d * d, mstat, preferred_element_type=jnp.float32)
    return d * lax.rsqrt(var + 1e-5) * gamma + beta


def _mlp_body(g_ref, ea_ref, w1b_ref, mstat_ref, b1_ref, g1_ref, beta1_ref,
              w2_ref, b2_ref, g2_ref, beta2_ref, o_ref):
    mstat = mstat_ref[...]
    ea = jnp.dot(ea_ref[...], w1b_ref[...],
                 preferred_element_type=jnp.float32)
    h = jnp.maximum(g_ref[...] + ea + b1_ref[...], 0.0)
    h = _ln2(h, mstat, g1_ref[...], beta1_ref[...])
    h = jnp.dot(h, w2_ref[...],
                preferred_element_type=jnp.float32) + b2_ref[...]
    h = jnp.maximum(h, 0.0)
    o_ref[...] = _ln2(h, mstat, g2_ref[...], beta2_ref[...])


def _edge_mlp(g2, ea2, w1b_blk, mstat, b1_blk, g1_blk, beta1_blk,
              w2_blk, b2_blk, g2_blk, beta2_blk):
    vec = pl.BlockSpec((1, 2 * HIDDEN), lambda i: (0, 0))
    return pl.pallas_call(
        _mlp_body,
        grid=(N_E2 // BLOCK_R,),
        in_specs=[
            pl.BlockSpec((BLOCK_R, 2 * HIDDEN), lambda i: (i, 0)),
            pl.BlockSpec((BLOCK_R, 2 * D_EDGE), lambda i: (i, 0)),
            pl.BlockSpec((2 * D_EDGE, 2 * HIDDEN), lambda i: (0, 0)),
            pl.BlockSpec((2 * HIDDEN, 2 * HIDDEN), lambda i: (0, 0)),
            vec, vec, vec,
            pl.BlockSpec((2 * HIDDEN, 2 * HIDDEN), lambda i: (0, 0)),
            vec, vec, vec,
        ],
        out_specs=pl.BlockSpec((BLOCK_R, 2 * HIDDEN), lambda i: (i, 0)),
        out_shape=jax.ShapeDtypeStruct((N_E2, 2 * HIDDEN), jnp.float32),
    )(g2, ea2, w1b_blk, mstat, b1_blk, g1_blk, beta1_blk,
      w2_blk, b2_blk, g2_blk, beta2_blk)


SUP_CH = 4
SUP_E = SUP_CH * CHUNK
SUPERS_PER_TILE = CHUNKS_PER_TILE // SUP_CH


def _scatter_body(h2_hbm, dst2_hbm, zeros_hbm, out_hbm,
                  idx2, big0, big1, ls0, ls1, accum):
    core = lax.axis_index("c")
    sid = lax.axis_index("s")
    wid = sid * NUM_CORES + core
    crow = wid * CHUNKS_PER_TILE
    base = wid * EDGES_PER_TILE
    stripe = sid * ROWS_PER_SUBCORE
    bigs = (big0, big1)
    lsems = (ls0, ls1)

    pltpu.sync_copy(zeros_hbm, accum.at[pl.ds(stripe, ROWS_PER_SUBCORE)])
    pltpu.sync_copy(dst2_hbm.at[pl.ds(crow, CHUNKS_PER_TILE)], idx2)
    plsc.subcore_barrier()

    @pl.loop(0, SUPERS_PER_TILE // 2)
    def _(p):
        loads = []
        for b in range(2):
            s = p * 2 + b
            loads.append(pltpu.async_copy(
                h2_hbm.at[pl.ds(base + s * SUP_E, SUP_E)], bigs[b], lsems[b]))
        for b in range(2):
            s = p * 2 + b
            loads[b].wait()
            for j in range(SUP_CH):
                pltpu.sync_copy(bigs[b].at[pl.ds(j * CHUNK, CHUNK)],
                                accum.at[idx2.at[s * SUP_CH + j]], add=True)

    plsc.subcore_barrier()
    pltpu.sync_copy(accum.at[pl.ds(stripe, ROWS_PER_SUBCORE)],
                    out_hbm.at[pl.ds(core * N_ACC + stripe,
                                     ROWS_PER_SUBCORE)])


def _sc_scatter_add(h2, dst2, zeros):
    k = pl.kernel(
        _scatter_body,
        out_type=jax.ShapeDtypeStruct((NUM_CORES * N_ACC, HIDDEN),
                                      jnp.float32),
        mesh=plsc.VectorSubcoreMesh(**_SC_MESH),
        scratch_types=[
            pltpu.VMEM((CHUNKS_PER_TILE, CHUNK), jnp.int32),
            pltpu.VMEM((SUP_E, HIDDEN), jnp.float32),
            pltpu.VMEM((SUP_E, HIDDEN), jnp.float32),
            pltpu.SemaphoreType.DMA,
            pltpu.SemaphoreType.DMA,
            pltpu.VMEM_SHARED((N_ACC, HIDDEN), jnp.float32),
        ],
        compiler_params=_SC_PARAMS,
    )
    return k(h2, dst2, zeros)


def _sum_body(p_ref, o_ref):
    o_ref[...] = p_ref[0] + p_ref[1]


def _sum_partials(partials):
    return pl.pallas_call(
        _sum_body,
        out_shape=jax.ShapeDtypeStruct((N_NODES, HIDDEN), jnp.float32),
    )(partials.reshape(NUM_CORES, N_ACC, HIDDEN))


def _blockdiag2(w):
    k, n = w.shape
    z = jnp.zeros((k, n), w.dtype)
    return jnp.concatenate([jnp.concatenate([w, z], axis=1),
                            jnp.concatenate([z, w], axis=1)], axis=0)


def _dup(v):
    return jnp.concatenate([v, v]).reshape(1, 2 * HIDDEN)


def kernel(x, edge_index, edge_attr, W1, b1, g1, beta1, W2, b2, g2, beta2):
    src = edge_index[0]
    dst = edge_index[1]
    ea_p = edge_attr
    w1a = W1[:D_FEAT]
    w1b = W1[D_FEAT:]
    zeros = jnp.zeros((ROWS_PER_SUBCORE, HIDDEN), jnp.float32)

    xa = _compute_xa(x, w1a)
    g = _sc_gather(xa, src.reshape(NUM_CHUNK_ROWS, CHUNK))
    mstat = _blockdiag2(jnp.full((HIDDEN, HIDDEN), _INV_H, jnp.float32))
    h2 = _edge_mlp(g.reshape(N_E2, 2 * HIDDEN),
                   ea_p.reshape(N_E2, 2 * D_EDGE),
                   _blockdiag2(w1b), mstat, _dup(b1), _dup(g1), _dup(beta1),
                   _blockdiag2(W2), _dup(b2), _dup(g2), _dup(beta2))
    partials = _sc_scatter_add(h2.reshape(N_EDGES, HIDDEN),
                               dst.reshape(NUM_CHUNK_ROWS, CHUNK), zeros)
    return _sum_partials(partials)
